# baseline (speedup 1.0000x reference)
import functools

import jax
import jax.numpy as jnp
from jax import lax
from jax.experimental import pallas as pl
from jax.experimental.pallas import tpu as pltpu
from jax.experimental.pallas import tpu_sc as plsc

D = 64
B_POS = 16384
B_NEG = 81920
P = B_POS + B_NEG

NC = 2
NS = 16
NW = NC * NS
B_PER_W = P // NW
CHUNK = 512
N_CHUNKS = B_PER_W // CHUNK
LANES = 16


def _sc_scores(u_hbm, v_hbm, iu_hbm, iv_hbm, out_hbm,
               idx_u, idx_v, rows_u, rows_v, sc_v, sem_u, sem_v):
    wid = lax.axis_index("s") * NC + lax.axis_index("c")
    base = wid * B_PER_W

    def chunk_body(c, carry):
        off = base + c * CHUNK
        pltpu.sync_copy(iu_hbm.at[pl.ds(off, CHUNK)], idx_u)
        pltpu.sync_copy(iv_hbm.at[pl.ds(off, CHUNK)], idx_v)
        cp_u = pltpu.async_copy(u_hbm.at[idx_u], rows_u, sem_u)
        cp_v = pltpu.async_copy(v_hbm.at[idx_v], rows_v, sem_v)
        cp_u.wait()
        cp_v.wait()

        def block_body(b, carry2):
            rids = b * LANES + lax.iota(jnp.int32, 16)
            acc = jnp.zeros((LANES,), jnp.float32)
            for d in range(D):
                cid = jnp.full((LANES,), d, jnp.int32)
                ul = plsc.load_gather(rows_u, [rids, cid])
                vl = plsc.load_gather(rows_v, [rids, cid])
                acc = acc + ul * vl
            sc_v[pl.ds(b * LANES, LANES)] = acc
            return carry2

        lax.fori_loop(0, CHUNK // LANES, block_body, 0)
        pltpu.sync_copy(sc_v, out_hbm.at[pl.ds(off, CHUNK)])
        return carry

    lax.fori_loop(0, N_CHUNKS, chunk_body, 0)


def _tc_loss_body(s_ref, o_ref):
    s = s_ref[...]
    row = lax.broadcasted_iota(jnp.int32, s.shape, 0)
    x = jnp.where(row < B_POS // 128, s, -s)
    ls = jnp.minimum(x, 0.0) - jnp.log1p(jnp.exp(-jnp.abs(x)))
    o_ref[0, 0] = -jnp.sum(ls)


def kernel(pos_u, pos_v, neg_u, neg_v, u_weight, v_weight):
    all_u = jnp.concatenate([pos_u, neg_u])
    all_v = jnp.concatenate([pos_v, neg_v])

    mesh = plsc.VectorSubcoreMesh(core_axis_name="c", subcore_axis_name="s")
    sc_fn = functools.partial(
        pl.kernel,
        out_type=jax.ShapeDtypeStruct((P,), jnp.float32),
        mesh=mesh,
        scratch_types=[
            pltpu.VMEM((CHUNK,), jnp.int32),
            pltpu.VMEM((CHUNK,), jnp.int32),
            pltpu.VMEM((CHUNK, D), jnp.float32),
            pltpu.VMEM((CHUNK, D), jnp.float32),
            pltpu.VMEM((CHUNK,), jnp.float32),
            pltpu.SemaphoreType.DMA,
            pltpu.SemaphoreType.DMA,
        ],
        compiler_params=pltpu.CompilerParams(
            needs_layout_passes=False, use_tc_tiling_on_sc=False),
    )(_sc_scores)
    scores = sc_fn(u_weight, v_weight, all_u, all_v)

    loss = pl.pallas_call(
        _tc_loss_body,
        out_shape=jax.ShapeDtypeStruct((1, 1), jnp.float32),
        out_specs=pl.BlockSpec(memory_space=pltpu.SMEM),
    )(scores.reshape(P // 128, 128))
    return loss[0, 0]


# speedup vs baseline: 1.0482x; 1.0482x over previous
import functools

import jax
import jax.numpy as jnp
from jax import lax
from jax.experimental import pallas as pl
from jax.experimental.pallas import tpu as pltpu
from jax.experimental.pallas import tpu_sc as plsc

D = 64
B_POS = 16384
B_NEG = 81920
P = B_POS + B_NEG

NC = 2
NS = 16
NW = NC * NS
B_PER_W = P // NW
CHUNK = 384
N_CHUNKS = B_PER_W // CHUNK
LANES = 16


def _sc_scores(u_hbm, v_hbm, iu_hbm, iv_hbm, out_hbm,
               idx_u, idx_v, rows_u, rows_v, sc_v,
               sem_iu, sem_iv, sem_u, sem_v, sem_out):
    wid = lax.axis_index("s") * NC + lax.axis_index("c")
    base = wid * B_PER_W

    def issue_idx(c):
        k = c % 2
        off = base + c * CHUNK
        pltpu.async_copy(iu_hbm.at[pl.ds(off, CHUNK)], idx_u.at[k], sem_iu.at[k])
        pltpu.async_copy(iv_hbm.at[pl.ds(off, CHUNK)], idx_v.at[k], sem_iv.at[k])

    def wait_idx(c):
        k = c % 2
        pltpu.make_async_copy(iu_hbm.at[pl.ds(0, CHUNK)], idx_u.at[k], sem_iu.at[k]).wait()
        pltpu.make_async_copy(iv_hbm.at[pl.ds(0, CHUNK)], idx_v.at[k], sem_iv.at[k]).wait()

    def issue_gather(c):
        k = c % 2
        pltpu.async_copy(u_hbm.at[idx_u.at[k]], rows_u.at[k], sem_u.at[k])
        pltpu.async_copy(v_hbm.at[idx_v.at[k]], rows_v.at[k], sem_v.at[k])

    def wait_gather(c):
        k = c % 2
        pltpu.make_async_copy(u_hbm.at[idx_u.at[k]], rows_u.at[k], sem_u.at[k]).wait()
        pltpu.make_async_copy(v_hbm.at[idx_v.at[k]], rows_v.at[k], sem_v.at[k]).wait()

    issue_idx(0)
    issue_idx(1)
    wait_idx(0)
    issue_gather(0)

    for c in range(N_CHUNKS):
        k = c % 2
        wait_gather(c)
        if c + 1 < N_CHUNKS:
            wait_idx(c + 1)
            issue_gather(c + 1)
        if c + 2 < N_CHUNKS:
            issue_idx(c + 2)

        ru = rows_u.at[k]
        rv = rows_v.at[k]

        def block_body(b, carry2, ru=ru, rv=rv, k=k):
            rids = b * LANES + lax.iota(jnp.int32, 16)
            acc = jnp.zeros((LANES,), jnp.float32)
            for d in range(D):
                cid = jnp.full((LANES,), d, jnp.int32)
                ul = plsc.load_gather(ru, [rids, cid])
                vl = plsc.load_gather(rv, [rids, cid])
                acc = acc + ul * vl
            sc_v[k, pl.ds(b * LANES, LANES)] = acc
            return carry2

        lax.fori_loop(0, CHUNK // LANES, block_body, 0)
        if c >= 2:
            pltpu.make_async_copy(
                sc_v.at[k], out_hbm.at[pl.ds(0, CHUNK)], sem_out.at[k]).wait()
        pltpu.async_copy(sc_v.at[k], out_hbm.at[pl.ds(base + c * CHUNK, CHUNK)],
                         sem_out.at[k])

    for c in (N_CHUNKS - 2, N_CHUNKS - 1):
        k = c % 2
        pltpu.make_async_copy(
            sc_v.at[k], out_hbm.at[pl.ds(0, CHUNK)], sem_out.at[k]).wait()


def _tc_loss_body(s_ref, o_ref):
    s = s_ref[...]
    row = lax.broadcasted_iota(jnp.int32, s.shape, 0)
    x = jnp.where(row < B_POS // 128, s, -s)
    ls = jnp.minimum(x, 0.0) - jnp.log1p(jnp.exp(-jnp.abs(x)))
    o_ref[0, 0] = -jnp.sum(ls)


def kernel(pos_u, pos_v, neg_u, neg_v, u_weight, v_weight):
    all_u = jnp.concatenate([pos_u, neg_u])
    all_v = jnp.concatenate([pos_v, neg_v])

    mesh = plsc.VectorSubcoreMesh(core_axis_name="c", subcore_axis_name="s")
    sc_fn = functools.partial(
        pl.kernel,
        out_type=jax.ShapeDtypeStruct((P,), jnp.float32),
        mesh=mesh,
        scratch_types=[
            pltpu.VMEM((2, CHUNK), jnp.int32),
            pltpu.VMEM((2, CHUNK), jnp.int32),
            pltpu.VMEM((2, CHUNK, D), jnp.float32),
            pltpu.VMEM((2, CHUNK, D), jnp.float32),
            pltpu.VMEM((2, CHUNK), jnp.float32),
            pltpu.SemaphoreType.DMA((2,)),
            pltpu.SemaphoreType.DMA((2,)),
            pltpu.SemaphoreType.DMA((2,)),
            pltpu.SemaphoreType.DMA((2,)),
            pltpu.SemaphoreType.DMA((2,)),
        ],
        compiler_params=pltpu.CompilerParams(
            needs_layout_passes=False, use_tc_tiling_on_sc=False),
    )(_sc_scores)
    scores = sc_fn(u_weight, v_weight, all_u, all_v)

    loss = pl.pallas_call(
        _tc_loss_body,
        out_shape=jax.ShapeDtypeStruct((1, 1), jnp.float32),
        out_specs=pl.BlockSpec(memory_space=pltpu.SMEM),
    )(scores.reshape(P // 128, 128))
    return loss[0, 0]


# speedup vs baseline: 1.6710x; 1.5941x over previous
import functools

import jax
import jax.numpy as jnp
from jax import lax
from jax.experimental import pallas as pl
from jax.experimental.pallas import tpu as pltpu
from jax.experimental.pallas import tpu_sc as plsc

D = 64
B_POS = 16384
B_NEG = 81920
P = B_POS + B_NEG

NC = 2
NS = 16
NW = NC * NS
B_PER_W = P // NW
CHUNK = 384
N_CHUNKS = B_PER_W // CHUNK
LANES = 16


def _sc_scores(u_hbm, v_hbm, iu_hbm, iv_hbm, out_hbm,
               idx_u, idx_v, rows_u, rows_v, sc_v,
               sem_iu, sem_iv, sem_u, sem_v, sem_out):
    wid = lax.axis_index("s") * NC + lax.axis_index("c")
    base = wid * B_PER_W

    def issue_idx(c):
        k = c % 2
        off = base + c * CHUNK
        pltpu.async_copy(iu_hbm.at[pl.ds(off, CHUNK)], idx_u.at[k], sem_iu.at[k])
        pltpu.async_copy(iv_hbm.at[pl.ds(off, CHUNK)], idx_v.at[k], sem_iv.at[k])

    def wait_idx(c):
        k = c % 2
        pltpu.make_async_copy(iu_hbm.at[pl.ds(0, CHUNK)], idx_u.at[k], sem_iu.at[k]).wait()
        pltpu.make_async_copy(iv_hbm.at[pl.ds(0, CHUNK)], idx_v.at[k], sem_iv.at[k]).wait()

    def issue_gather(c):
        k = c % 2
        pltpu.async_copy(u_hbm.at[idx_u.at[k]], rows_u.at[k], sem_u.at[k])
        pltpu.async_copy(v_hbm.at[idx_v.at[k]], rows_v.at[k], sem_v.at[k])

    def wait_gather(c):
        k = c % 2
        pltpu.make_async_copy(u_hbm.at[idx_u.at[k]], rows_u.at[k], sem_u.at[k]).wait()
        pltpu.make_async_copy(v_hbm.at[idx_v.at[k]], rows_v.at[k], sem_v.at[k]).wait()

    issue_idx(0)
    issue_idx(1)
    wait_idx(0)
    issue_gather(0)

    for c in range(N_CHUNKS):
        k = c % 2
        wait_gather(c)
        if c + 1 < N_CHUNKS:
            wait_idx(c + 1)
            issue_gather(c + 1)
        if c + 2 < N_CHUNKS:
            issue_idx(c + 2)

        ru = rows_u.at[k]
        rv = rows_v.at[k]

        def block_body(b, carry2, ru=ru, rv=rv, k=k):
            rids = b * LANES + lax.iota(jnp.int32, 16)
            lane = lax.iota(jnp.int32, 16)
            acc = jnp.zeros((LANES,), jnp.float32)
            for d in range(D):
                cid = (lane + d) & (D - 1)
                ul = plsc.load_gather(ru, [rids, cid])
                vl = plsc.load_gather(rv, [rids, cid])
                acc = acc + ul * vl
            sc_v[k, pl.ds(b * LANES, LANES)] = acc
            return carry2

        lax.fori_loop(0, CHUNK // LANES, block_body, 0)
        if c >= 2:
            pltpu.make_async_copy(
                sc_v.at[k], out_hbm.at[pl.ds(0, CHUNK)], sem_out.at[k]).wait()
        pltpu.async_copy(sc_v.at[k], out_hbm.at[pl.ds(base + c * CHUNK, CHUNK)],
                         sem_out.at[k])

    for c in (N_CHUNKS - 2, N_CHUNKS - 1):
        k = c % 2
        pltpu.make_async_copy(
            sc_v.at[k], out_hbm.at[pl.ds(0, CHUNK)], sem_out.at[k]).wait()


def _tc_loss_body(s_ref, o_ref):
    s = s_ref[...]
    row = lax.broadcasted_iota(jnp.int32, s.shape, 0)
    x = jnp.where(row < B_POS // 128, s, -s)
    ls = jnp.minimum(x, 0.0) - jnp.log1p(jnp.exp(-jnp.abs(x)))
    o_ref[0, 0] = -jnp.sum(ls)


def kernel(pos_u, pos_v, neg_u, neg_v, u_weight, v_weight):
    all_u = jnp.concatenate([pos_u, neg_u])
    all_v = jnp.concatenate([pos_v, neg_v])

    mesh = plsc.VectorSubcoreMesh(core_axis_name="c", subcore_axis_name="s")
    sc_fn = functools.partial(
        pl.kernel,
        out_type=jax.ShapeDtypeStruct((P,), jnp.float32),
        mesh=mesh,
        scratch_types=[
            pltpu.VMEM((2, CHUNK), jnp.int32),
            pltpu.VMEM((2, CHUNK), jnp.int32),
            pltpu.VMEM((2, CHUNK, D), jnp.float32),
            pltpu.VMEM((2, CHUNK, D), jnp.float32),
            pltpu.VMEM((2, CHUNK), jnp.float32),
            pltpu.SemaphoreType.DMA((2,)),
            pltpu.SemaphoreType.DMA((2,)),
            pltpu.SemaphoreType.DMA((2,)),
            pltpu.SemaphoreType.DMA((2,)),
            pltpu.SemaphoreType.DMA((2,)),
        ],
        compiler_params=pltpu.CompilerParams(
            needs_layout_passes=False, use_tc_tiling_on_sc=False),
    )(_sc_scores)
    scores = sc_fn(u_weight, v_weight, all_u, all_v)

    loss = pl.pallas_call(
        _tc_loss_body,
        out_shape=jax.ShapeDtypeStruct((1, 1), jnp.float32),
        out_specs=pl.BlockSpec(memory_space=pltpu.SMEM),
    )(scores.reshape(P // 128, 128))
    return loss[0, 0]
